# baseline (speedup 1.0000x reference)
import functools

import jax
import jax.numpy as jnp
import numpy as np
from jax.experimental import pallas as pl
from jax.experimental.pallas import tpu as pltpu
from jax.experimental.pallas import tpu_sc as plsc


def _proj_body(h, easum, We_l, Wl, bl, Wr, br, attf, Pbd, Esel, inv_e,
               xl_ref, xr_ref, accn_ref, accd_ref):
    xl = h @ Wl + bl
    xr = h @ Wr + br
    em = (easum * inv_e) @ We_l
    t = xl + xr + em
    m = jnp.maximum(t, 0.2 * t)
    s = (m * attf) @ Pbd
    w = jnp.exp(s)
    xl_ref[...] = xl
    xr_ref[...] = xr
    accn_ref[...] = w * xl
    accd_ref[...] = w @ Esel


def _in_proj_kernel(x_ref, Win_ref, bin_ref, easum_ref, We_ref, Wl_ref,
                    bl_ref, Wr_ref, br_ref, attf_ref, Pbd_ref, Esel_ref,
                    h_ref, xl_ref, xr_ref, accn_ref, accd_ref, *, inv_e):
    h = x_ref[...] @ Win_ref[...] + bin_ref[...]
    h_ref[...] = h
    _proj_body(h, easum_ref[...], We_ref[0], Wl_ref[...], bl_ref[...],
               Wr_ref[...], br_ref[...], attf_ref[...], Pbd_ref[...],
               Esel_ref[...], inv_e, xl_ref, xr_ref, accn_ref, accd_ref)


def _e_kernel(ea_ref, We_ref, out_ref):
    out_ref[0] = ea_ref[...] @ We_ref[0]


def _ea_sum_kernel(ea_ref, out_ref):
    @pl.when(pl.program_id(0) == 0)
    def _():
        out_ref[...] = jnp.zeros_like(out_ref)
    out_ref[...] += jnp.sum(ea_ref[...], axis=0, keepdims=True)


def _segnorm_kernel(num_ref, den_ref, bias_ref, Bh_ref, out_ref, sums_ref,
                    s1, s2, *, nblk):
    num = num_ref[0] + num_ref[1]
    den = (den_ref[0] + den_ref[1]) @ Bh_ref[...]
    out = num / (den + 1e-16) + bias_ref[...]
    out_ref[...] = out

    @pl.when(pl.program_id(0) == 0)
    def _():
        s1[...] = jnp.zeros_like(s1)
        s2[...] = jnp.zeros_like(s2)

    s1[...] += jnp.sum(out, axis=0, keepdims=True)
    s2[...] += jnp.sum(out * out, axis=0, keepdims=True)

    @pl.when(pl.program_id(0) == nblk - 1)
    def _():
        sums_ref[...] = jnp.concatenate([s1[...], s2[...]], axis=0)


def _graphnorm(out, sums, gnw, gnb, gnms, hprev, inv_n):
    mean = sums[0:1] * inv_n
    ex2 = sums[1:2] * inv_n
    var = ex2 - (2.0 * gnms - gnms * gnms) * mean * mean
    outc = out - gnms * mean
    outn = gnw * outc / jnp.sqrt(var + 1e-5) + gnb
    return jnp.maximum(outn, 0.0) + 0.1 * hprev


def _norm_proj_kernel(out_ref, sums_ref, hprev_ref, gnw_ref, gnb_ref,
                      gnms_ref, easum_ref, We_ref, Wl_ref, bl_ref, Wr_ref,
                      br_ref, attf_ref, Pbd_ref, Esel_ref,
                      h_ref, xl_ref, xr_ref, accn_ref, accd_ref,
                      *, inv_n, inv_e):
    h = _graphnorm(out_ref[...], sums_ref[...], gnw_ref[...], gnb_ref[...],
                   gnms_ref[...], hprev_ref[...], inv_n)
    h_ref[...] = h
    _proj_body(h, easum_ref[...], We_ref[0], Wl_ref[...], bl_ref[...],
               Wr_ref[...], br_ref[...], attf_ref[...], Pbd_ref[...],
               Esel_ref[...], inv_e, xl_ref, xr_ref, accn_ref, accd_ref)


def _norm_head_kernel(out_ref, sums_ref, hprev_ref, gnw_ref, gnb_ref,
                      gnms_ref, Wp1_ref, bp1_ref, Wp2_ref, bp2_ref,
                      res_ref, hsum, *, inv_n, nblk):
    h = _graphnorm(out_ref[...], sums_ref[...], gnw_ref[...], gnb_ref[...],
                   gnms_ref[...], hprev_ref[...], inv_n)

    @pl.when(pl.program_id(0) == 0)
    def _():
        hsum[...] = jnp.zeros_like(hsum)

    hsum[...] += jnp.sum(h, axis=0, keepdims=True)

    @pl.when(pl.program_id(0) == nblk - 1)
    def _():
        g = hsum[...] * inv_n
        z = jnp.maximum(g @ Wp1_ref[...] + bp1_ref[...], 0.0)
        res_ref[...] = z @ Wp2_ref[...] + bp2_ref[...]


_CH = 16
_NRING = 5
_NWORKERS = 32


def _edge_sc_body(layer, n, e_cnt, hc, H, C,
                  src_hbm, dst_hbm, xl_hbm, xr_hbm, e_hbm, att_hbm,
                  initn_hbm, initd_hbm, zn_hbm, zd_hbm,
                  outn_hbm, outd_hbm,
                  srcv, dstv, glv, grv, ev, denv, attv, accn, accd,
                  sem_g, sem_e, sem_s):
    c = jax.lax.axis_index("c")
    s = jax.lax.axis_index("s")
    wid = s * 2 + c
    rpt = n // 16
    r0 = s * rpt

    @pl.when(c == 0)
    def _():
        pltpu.sync_copy(initn_hbm.at[pl.ds(r0, rpt)], accn.at[pl.ds(r0, rpt)])
        pltpu.sync_copy(initd_hbm.at[pl.ds(r0, rpt)], accd.at[pl.ds(r0, rpt)])

    @pl.when(c == 1)
    def _():
        pltpu.sync_copy(zn_hbm.at[pl.ds(r0, rpt)], accn.at[pl.ds(r0, rpt)])
        pltpu.sync_copy(zd_hbm.at[pl.ds(r0, rpt)], accd.at[pl.ds(r0, rpt)])

    pltpu.sync_copy(att_hbm, attv)
    plsc.subcore_barrier()

    per_w = e_cnt // _NWORKERS
    base_w = wid * per_w
    iota16 = jax.lax.iota(jnp.int32, C)

    nchunks = per_w // _CH
    NR = _NRING

    def _fetch(kk, sl):
        base = base_w + kk * _CH
        pltpu.sync_copy(src_hbm.at[pl.ds(base, _CH)], srcv.at[sl])
        pltpu.sync_copy(dst_hbm.at[pl.ds(base, _CH)], dstv.at[sl])
        pltpu.async_copy(xl_hbm.at[srcv.at[sl]], glv.at[sl], sem_g.at[sl])
        pltpu.async_copy(xr_hbm.at[dstv.at[sl]], grv.at[sl], sem_g.at[sl])
        pltpu.async_copy(e_hbm.at[layer, pl.ds(base, _CH)], ev.at[sl],
                         sem_e.at[sl])

    def _drain_scatter(sl):
        pltpu.make_async_copy(glv.at[sl], accn.at[dstv.at[sl]],
                              sem_s.at[sl]).wait()
        pltpu.make_async_copy(denv.at[sl], accd.at[dstv.at[sl]],
                              sem_s.at[sl]).wait()

    def _wait_inputs(sl):
        pltpu.make_async_copy(xl_hbm.at[srcv.at[sl]], glv.at[sl],
                              sem_g.at[sl]).wait()
        pltpu.make_async_copy(xr_hbm.at[dstv.at[sl]], grv.at[sl],
                              sem_g.at[sl]).wait()
        pltpu.make_async_copy(e_hbm.at[layer, pl.ds(base_w, _CH)], ev.at[sl],
                              sem_e.at[sl]).wait()

    def _compute(sl):
        @plsc.parallel_loop(0, _CH, unroll=8)
        def _edge(i):
            gl = [glv[sl, i, pl.ds(h * C, C)] for h in range(H)]
            gr = [grv[sl, i, pl.ds(h * C, C)] for h in range(H)]
            eh = [ev[sl, i, pl.ds(h * C, C)] for h in range(H)]
            wv = []
            for h in range(H):
                t = gl[h] + gr[h] + eh[h]
                m = jnp.maximum(t, 0.2 * t)
                alpha = jnp.sum(m * attv[h])
                wv.append(jnp.exp(jax.lax.broadcast_in_dim(alpha, (C,), ())))
            den_parts = [jnp.where(iota16 == h, wv[h], 0.0) for h in range(H)]
            while len(den_parts) > 1:
                den_parts = [a + b for a, b in
                             zip(den_parts[::2], den_parts[1::2])]
            for h in range(H):
                glv[sl, i, pl.ds(h * C, C)] = gl[h] * wv[h]
            denv[sl, i, :] = den_parts[0]

    _fetch(0, 0)

    @pl.loop(0, nchunks // NR)
    def _group(j):
        k0 = j * NR
        for sl in range(NR):
            k = k0 + sl
            sln = (sl + 1) % NR

            @pl.when(k + 1 < nchunks)
            def _():
                @pl.when(k + 1 >= NR)
                def _():
                    _drain_scatter(sln)
                _fetch(k + 1, sln)

            _wait_inputs(sl)
            _compute(sl)
            pltpu.async_copy(glv.at[sl], accn.at[dstv.at[sl]],
                             sem_s.at[sl], add=True)
            pltpu.async_copy(denv.at[sl], accd.at[dstv.at[sl]],
                             sem_s.at[sl], add=True)

    for sl in range(NR):
        _drain_scatter(sl)
    plsc.subcore_barrier()
    pltpu.sync_copy(accn.at[pl.ds(r0, rpt)], outn_hbm.at[c, pl.ds(r0, rpt)])
    pltpu.sync_copy(accd.at[pl.ds(r0, rpt)], outd_hbm.at[c, pl.ds(r0, rpt)])


def _edge_pass(xl, xr, e_all, layer, src, dst, init_num, init_den,
               zeros_num, zeros_den, att_l, n, H, C):
    hc = H * C
    e_cnt = src.shape[0]
    f32 = jnp.float32
    mesh = plsc.VectorSubcoreMesh(core_axis_name="c", subcore_axis_name="s")

    edge_kernel = functools.partial(
        pl.kernel,
        out_type=[jax.ShapeDtypeStruct((2, n, hc), f32),
                  jax.ShapeDtypeStruct((2, n, C), f32)],
        mesh=mesh,
        scratch_types=[
            pltpu.VMEM((_NRING, _CH), jnp.int32),
            pltpu.VMEM((_NRING, _CH), jnp.int32),
            pltpu.VMEM((_NRING, _CH, hc), f32),
            pltpu.VMEM((_NRING, _CH, hc), f32),
            pltpu.VMEM((_NRING, _CH, hc), f32),
            pltpu.VMEM((_NRING, _CH, C), f32),
            pltpu.VMEM((H, C), f32),
            pltpu.VMEM_SHARED((n, hc), f32),
            pltpu.VMEM_SHARED((n, C), f32),
            pltpu.SemaphoreType.DMA((_NRING,)),
            pltpu.SemaphoreType.DMA((_NRING,)),
            pltpu.SemaphoreType.DMA((_NRING,)),
        ],
        compiler_params=pltpu.CompilerParams(
            use_tc_tiling_on_sc=False, needs_layout_passes=False),
    )(functools.partial(_edge_sc_body, layer, n, e_cnt, hc, H, C))

    return edge_kernel(src, dst, xl, xr, e_all, att_l, init_num, init_den,
                       zeros_num, zeros_den)


_BN = 400


def _full(shape):
    return pl.BlockSpec(shape, lambda *args: tuple(0 for _ in shape))


def _nodeblk(width):
    return pl.BlockSpec((_BN, width), lambda b: (b, 0))


def _pairblk(width):
    return pl.BlockSpec((2, _BN, width), lambda b: (0, b, 0))


def kernel(x, edge_index, edge_attr, W_in, b_in, Wl, bl, Wr, br, We, att,
           bias_conv, gn_w, gn_b, gn_ms, Wp1, bp1, Wp2, bp2):
    n, d = x.shape
    e_cnt, de = edge_attr.shape
    L, _, hc = Wl.shape
    H, C = att.shape[1], att.shape[2]
    nblk = n // _BN
    inv_n = 1.0 / n
    inv_e = 1.0 / e_cnt

    f32 = jnp.float32
    ii = np.arange(hc)
    Pbd = jnp.asarray((ii[:, None] // C == ii[None, :] // C), dtype=f32)
    Esel = jnp.zeros((hc, C), f32).at[np.arange(H) * C, np.arange(H)].set(1.0)
    Bh = jnp.asarray((np.arange(C)[:, None] == ii[None, :] // C), dtype=f32)

    src = edge_index[0]
    dst = edge_index[1]

    eblk = 4000
    ea_sum = pl.pallas_call(
        _ea_sum_kernel,
        grid=(e_cnt // eblk,),
        in_specs=[pl.BlockSpec((eblk, de), lambda b: (b, 0))],
        out_specs=pl.BlockSpec((1, de), lambda b: (0, 0)),
        out_shape=jax.ShapeDtypeStruct((1, de), f32),
    )(edge_attr)

    e_all = pl.pallas_call(
        _e_kernel,
        grid=(L, e_cnt // eblk),
        in_specs=[pl.BlockSpec((eblk, de), lambda l, b: (b, 0)),
                  pl.BlockSpec((1, de, hc), lambda l, b: (l, 0, 0))],
        out_specs=pl.BlockSpec((1, eblk, hc), lambda l, b: (l, b, 0)),
        out_shape=jax.ShapeDtypeStruct((L, e_cnt, hc), f32),
    )(edge_attr, We)

    weight_specs = [
        _full((1, de)),
        _full((1, de, hc)),
        _full((d, hc)),
        _full((1, hc)),
        _full((d, hc)),
        _full((1, hc)),
        _full((1, hc)),
        _full((hc, hc)),
        _full((hc, C)),
    ]
    proj_outs = (
        [jax.ShapeDtypeStruct((n, d), f32)] * 3
        + [jax.ShapeDtypeStruct((n, hc), f32),
           jax.ShapeDtypeStruct((n, C), f32)]
    )
    proj_out_specs = [_nodeblk(d)] * 3 + [_nodeblk(hc), _nodeblk(C)]

    h, xl, xr, init_num, init_den = pl.pallas_call(
        functools.partial(_in_proj_kernel, inv_e=inv_e),
        grid=(nblk,),
        in_specs=[_nodeblk(d), _full((d, d)), _full((1, d))] + weight_specs,
        out_specs=proj_out_specs,
        out_shape=proj_outs,
    )(x, W_in, b_in.reshape(1, d), ea_sum, We[0:1], Wl[0], bl[0:1], Wr[0],
      br[0:1], att[0].reshape(1, hc), Pbd, Esel)

    zeros_num = jnp.zeros((n, hc), f32)
    zeros_den = jnp.zeros((n, C), f32)
    for l in range(L):
        acc_num, acc_den = _edge_pass(xl, xr, e_all, l, src, dst, init_num,
                                      init_den, zeros_num, zeros_den,
                                      att[l], n, H, C)

        out_feat, sums = pl.pallas_call(
            functools.partial(_segnorm_kernel, nblk=nblk),
            grid=(nblk,),
            in_specs=[_pairblk(hc), _pairblk(C), _full((1, hc)),
                      _full((C, hc))],
            out_specs=[_nodeblk(hc), pl.BlockSpec((2, hc), lambda b: (0, 0))],
            out_shape=[jax.ShapeDtypeStruct((n, hc), f32),
                       jax.ShapeDtypeStruct((2, hc), f32)],
            scratch_shapes=[pltpu.VMEM((1, hc), f32)] * 2,
        )(acc_num, acc_den, bias_conv[l : l + 1], Bh)

        if l < L - 1:
            h, xl, xr, init_num, init_den = pl.pallas_call(
                functools.partial(_norm_proj_kernel, inv_n=inv_n, inv_e=inv_e),
                grid=(nblk,),
                in_specs=[_nodeblk(hc), pl.BlockSpec((2, hc), lambda b: (0, 0)),
                          _nodeblk(d), _full((1, d)), _full((1, d)),
                          _full((1, d))] + weight_specs,
                out_specs=proj_out_specs,
                out_shape=proj_outs,
            )(out_feat, sums, h, gn_w[l : l + 1], gn_b[l : l + 1],
              gn_ms[l : l + 1], ea_sum, We[l + 1 : l + 2], Wl[l + 1],
              bl[l + 1 : l + 2], Wr[l + 1], br[l + 1 : l + 2],
              att[l + 1].reshape(1, hc), Pbd, Esel)
        else:
            res = pl.pallas_call(
                functools.partial(_norm_head_kernel, inv_n=inv_n, nblk=nblk),
                grid=(nblk,),
                in_specs=[_nodeblk(hc), pl.BlockSpec((2, hc), lambda b: (0, 0)),
                          _nodeblk(d), _full((1, d)), _full((1, d)),
                          _full((1, d)), _full((d, d // 2)),
                          _full((1, d // 2)), _full((d // 2, 4)),
                          _full((1, 4))],
                out_specs=pl.BlockSpec((1, 4), lambda b: (0, 0)),
                out_shape=jax.ShapeDtypeStruct((1, 4), f32),
                scratch_shapes=[pltpu.VMEM((1, d), f32)],
            )(out_feat, sums, h, gn_w[l : l + 1], gn_b[l : l + 1],
              gn_ms[l : l + 1], Wp1, bp1.reshape(1, -1), Wp2,
              bp2.reshape(1, -1))
    return res


# speedup vs baseline: 1.6783x; 1.6783x over previous
import functools

import jax
import jax.numpy as jnp
import numpy as np
from jax.experimental import pallas as pl
from jax.experimental.pallas import tpu as pltpu
from jax.experimental.pallas import tpu_sc as plsc


def _proj_body(h, easum, We_l, Wl, bl, Wr, br, attf, Pbd, Esel, inv_e,
               xl_ref, xr_ref, accn_ref, accd_ref):
    xl = h @ Wl + bl
    xr = h @ Wr + br
    em = (easum * inv_e) @ We_l
    t = xl + xr + em
    m = jnp.maximum(t, 0.2 * t)
    s = (m * attf) @ Pbd
    w = jnp.exp(s)
    xl_ref[...] = xl
    xr_ref[...] = xr
    accn_ref[...] = w * xl
    accd_ref[...] = w @ Esel


def _in_proj_kernel(x_ref, Win_ref, bin_ref, easum_ref, We_ref, Wl_ref,
                    bl_ref, Wr_ref, br_ref, attf_ref, Pbd_ref, Esel_ref,
                    h_ref, xl_ref, xr_ref, accn_ref, accd_ref, *, inv_e):
    h = x_ref[...] @ Win_ref[...] + bin_ref[...]
    h_ref[...] = h
    _proj_body(h, easum_ref[...], We_ref[0], Wl_ref[...], bl_ref[...],
               Wr_ref[...], br_ref[...], attf_ref[...], Pbd_ref[...],
               Esel_ref[...], inv_e, xl_ref, xr_ref, accn_ref, accd_ref)


def _e_kernel(ea_ref, We_ref, out_ref):
    out_ref[...] = ea_ref[...] @ We_ref[0]


def _e0_kernel(ea_ref, We_ref, out_ref, sum_ref, s1, *, nblk):
    ea = ea_ref[...]
    out_ref[...] = ea @ We_ref[0]

    @pl.when(pl.program_id(0) == 0)
    def _():
        s1[...] = jnp.zeros_like(s1)

    s1[...] += jnp.sum(ea, axis=0, keepdims=True)

    @pl.when(pl.program_id(0) == nblk - 1)
    def _():
        sum_ref[...] = s1[...]


def _segnorm_kernel(num_ref, den_ref, bias_ref, Bh_ref, out_ref, sums_ref,
                    s1, s2, *, nblk):
    num = num_ref[0] + num_ref[1]
    den = (den_ref[0] + den_ref[1]) @ Bh_ref[...]
    out = num / (den + 1e-16) + bias_ref[...]
    out_ref[...] = out

    @pl.when(pl.program_id(0) == 0)
    def _():
        s1[...] = jnp.zeros_like(s1)
        s2[...] = jnp.zeros_like(s2)

    s1[...] += jnp.sum(out, axis=0, keepdims=True)
    s2[...] += jnp.sum(out * out, axis=0, keepdims=True)

    @pl.when(pl.program_id(0) == nblk - 1)
    def _():
        sums_ref[...] = jnp.concatenate([s1[...], s2[...]], axis=0)


def _graphnorm(out, sums, gnw, gnb, gnms, hprev, inv_n):
    mean = sums[0:1] * inv_n
    ex2 = sums[1:2] * inv_n
    var = ex2 - (2.0 * gnms - gnms * gnms) * mean * mean
    outc = out - gnms * mean
    outn = gnw * outc / jnp.sqrt(var + 1e-5) + gnb
    return jnp.maximum(outn, 0.0) + 0.1 * hprev


def _norm_proj_kernel(out_ref, sums_ref, hprev_ref, gnw_ref, gnb_ref,
                      gnms_ref, easum_ref, We_ref, Wl_ref, bl_ref, Wr_ref,
                      br_ref, attf_ref, Pbd_ref, Esel_ref,
                      h_ref, xl_ref, xr_ref, accn_ref, accd_ref,
                      *, inv_n, inv_e):
    h = _graphnorm(out_ref[...], sums_ref[...], gnw_ref[...], gnb_ref[...],
                   gnms_ref[...], hprev_ref[...], inv_n)
    h_ref[...] = h
    _proj_body(h, easum_ref[...], We_ref[0], Wl_ref[...], bl_ref[...],
               Wr_ref[...], br_ref[...], attf_ref[...], Pbd_ref[...],
               Esel_ref[...], inv_e, xl_ref, xr_ref, accn_ref, accd_ref)


def _norm_head_kernel(out_ref, sums_ref, hprev_ref, gnw_ref, gnb_ref,
                      gnms_ref, Wp1_ref, bp1_ref, Wp2_ref, bp2_ref,
                      res_ref, hsum, *, inv_n, nblk):
    h = _graphnorm(out_ref[...], sums_ref[...], gnw_ref[...], gnb_ref[...],
                   gnms_ref[...], hprev_ref[...], inv_n)

    @pl.when(pl.program_id(0) == 0)
    def _():
        hsum[...] = jnp.zeros_like(hsum)

    hsum[...] += jnp.sum(h, axis=0, keepdims=True)

    @pl.when(pl.program_id(0) == nblk - 1)
    def _():
        g = hsum[...] * inv_n
        z = jnp.maximum(g @ Wp1_ref[...] + bp1_ref[...], 0.0)
        res_ref[...] = z @ Wp2_ref[...] + bp2_ref[...]


_CH = 40
_NWORKERS = 32


def _edge_sc_body(n, e_cnt, hc, H, C,
                  src_hbm, dst_hbm, xl_hbm, xr_hbm, e_hbm, att_hbm,
                  initn_hbm, initd_hbm, zn_hbm, zd_hbm,
                  outn_hbm, outd_hbm,
                  srcv, dstv, glv, grv, ev, denv, attv, accn, accd,
                  sem_gl, sem_gr, sem_e, sem_s):
    c = jax.lax.axis_index("c")
    s = jax.lax.axis_index("s")
    wid = s * 2 + c
    rpt = n // 16
    r0 = s * rpt

    @pl.when(c == 0)
    def _():
        pltpu.sync_copy(initn_hbm.at[pl.ds(r0, rpt)], accn.at[pl.ds(r0, rpt)])
        pltpu.sync_copy(initd_hbm.at[pl.ds(r0, rpt)], accd.at[pl.ds(r0, rpt)])

    @pl.when(c == 1)
    def _():
        pltpu.sync_copy(zn_hbm.at[pl.ds(r0, rpt)], accn.at[pl.ds(r0, rpt)])
        pltpu.sync_copy(zd_hbm.at[pl.ds(r0, rpt)], accd.at[pl.ds(r0, rpt)])

    pltpu.sync_copy(att_hbm, attv)
    plsc.subcore_barrier()

    per_w = e_cnt // _NWORKERS
    base_w = wid * per_w
    iota16 = jax.lax.iota(jnp.int32, C)

    nchunks = per_w // _CH

    def _fetch(kk, rr, pp):
        base = base_w + kk * _CH
        pltpu.sync_copy(src_hbm.at[pl.ds(base, _CH)], srcv.at[pp])
        pltpu.sync_copy(dst_hbm.at[pl.ds(base, _CH)], dstv.at[rr])
        pltpu.async_copy(xl_hbm.at[srcv.at[pp]], glv.at[rr], sem_gl.at[rr])
        pltpu.async_copy(xr_hbm.at[dstv.at[rr]], grv.at[pp], sem_gr.at[pp])
        pltpu.async_copy(e_hbm.at[pl.ds(base, _CH)], ev.at[pp],
                         sem_e.at[pp])

    def _drain_scatter(rr):
        pltpu.make_async_copy(glv.at[rr], accn.at[dstv.at[rr]],
                              sem_s.at[rr]).wait()
        pltpu.make_async_copy(denv.at[rr], accd.at[dstv.at[rr]],
                              sem_s.at[rr]).wait()

    _fetch(0, 0, 0)

    @pl.loop(0, nchunks)
    def _chunk(k):
        r = k % 3
        p = k % 2
        rn = (k + 1) % 3
        pn = (k + 1) % 2

        @pl.when(k + 1 < nchunks)
        def _():
            @pl.when(k >= 2)
            def _():
                _drain_scatter(rn)
            _fetch(k + 1, rn, pn)

        pltpu.make_async_copy(xl_hbm.at[srcv.at[p]], glv.at[r],
                              sem_gl.at[r]).wait()
        pltpu.make_async_copy(xr_hbm.at[dstv.at[r]], grv.at[p],
                              sem_gr.at[p]).wait()
        pltpu.make_async_copy(e_hbm.at[pl.ds(base_w, _CH)], ev.at[p],
                              sem_e.at[p]).wait()

        @plsc.parallel_loop(0, _CH, unroll=8)
        def _edge(i):
            gl = [glv[r, i, pl.ds(h * C, C)] for h in range(H)]
            gr = [grv[p, i, pl.ds(h * C, C)] for h in range(H)]
            eh = [ev[p, i, pl.ds(h * C, C)] for h in range(H)]
            wv = []
            for h in range(H):
                t = gl[h] + gr[h] + eh[h]
                m = jnp.maximum(t, 0.2 * t)
                alpha = jnp.sum(m * attv[h])
                wv.append(jnp.exp(jax.lax.broadcast_in_dim(alpha, (C,), ())))
            den_parts = [jnp.where(iota16 == h, wv[h], 0.0) for h in range(H)]
            while len(den_parts) > 1:
                den_parts = [a + b for a, b in
                             zip(den_parts[::2], den_parts[1::2])]
            for h in range(H):
                glv[r, i, pl.ds(h * C, C)] = gl[h] * wv[h]
            denv[r, i, :] = den_parts[0]

        pltpu.async_copy(glv.at[r], accn.at[dstv.at[r]], sem_s.at[r],
                         add=True)
        pltpu.async_copy(denv.at[r], accd.at[dstv.at[r]], sem_s.at[r],
                         add=True)

    for t in (3, 2, 1):
        _drain_scatter((nchunks - t) % 3)
    plsc.subcore_barrier()
    pltpu.sync_copy(accn.at[pl.ds(r0, rpt)], outn_hbm.at[c, pl.ds(r0, rpt)])
    pltpu.sync_copy(accd.at[pl.ds(r0, rpt)], outd_hbm.at[c, pl.ds(r0, rpt)])


def _edge_pass(xl, xr, e_l, src, dst, init_num, init_den,
               zeros_num, zeros_den, att_l, n, H, C):
    hc = H * C
    e_cnt = src.shape[0]
    f32 = jnp.float32
    mesh = plsc.VectorSubcoreMesh(core_axis_name="c", subcore_axis_name="s")

    edge_kernel = functools.partial(
        pl.kernel,
        out_type=[jax.ShapeDtypeStruct((2, n, hc), f32),
                  jax.ShapeDtypeStruct((2, n, C), f32)],
        mesh=mesh,
        scratch_types=[
            pltpu.VMEM((2, _CH), jnp.int32),
            pltpu.VMEM((3, _CH), jnp.int32),
            pltpu.VMEM((3, _CH, hc), f32),
            pltpu.VMEM((2, _CH, hc), f32),
            pltpu.VMEM((2, _CH, hc), f32),
            pltpu.VMEM((3, _CH, C), f32),
            pltpu.VMEM((H, C), f32),
            pltpu.VMEM_SHARED((n, hc), f32),
            pltpu.VMEM_SHARED((n, C), f32),
            pltpu.SemaphoreType.DMA((3,)),
            pltpu.SemaphoreType.DMA((2,)),
            pltpu.SemaphoreType.DMA((2,)),
            pltpu.SemaphoreType.DMA((3,)),
        ],
        compiler_params=pltpu.CompilerParams(
            use_tc_tiling_on_sc=False, needs_layout_passes=False),
    )(functools.partial(_edge_sc_body, n, e_cnt, hc, H, C))

    return edge_kernel(src, dst, xl, xr, e_l, att_l, init_num, init_den,
                       zeros_num, zeros_den)


_BN = 400


def _full(shape):
    return pl.BlockSpec(shape, lambda *args: tuple(0 for _ in shape))


def _nodeblk(width):
    return pl.BlockSpec((_BN, width), lambda b: (b, 0))


def _pairblk(width):
    return pl.BlockSpec((2, _BN, width), lambda b: (0, b, 0))


def kernel(x, edge_index, edge_attr, W_in, b_in, Wl, bl, Wr, br, We, att,
           bias_conv, gn_w, gn_b, gn_ms, Wp1, bp1, Wp2, bp2):
    n, d = x.shape
    e_cnt, de = edge_attr.shape
    L, _, hc = Wl.shape
    H, C = att.shape[1], att.shape[2]
    nblk = n // _BN
    inv_n = 1.0 / n
    inv_e = 1.0 / e_cnt

    f32 = jnp.float32
    ii = np.arange(hc)
    Pbd = jnp.asarray((ii[:, None] // C == ii[None, :] // C), dtype=f32)
    Esel = jnp.zeros((hc, C), f32).at[np.arange(H) * C, np.arange(H)].set(1.0)
    Bh = jnp.asarray((np.arange(C)[:, None] == ii[None, :] // C), dtype=f32)

    src = edge_index[0]
    dst = edge_index[1]

    eblk = 8000
    nbe = e_cnt // eblk
    e0, ea_sum = pl.pallas_call(
        functools.partial(_e0_kernel, nblk=nbe),
        grid=(nbe,),
        in_specs=[pl.BlockSpec((eblk, de), lambda b: (b, 0)),
                  _full((1, de, hc))],
        out_specs=[pl.BlockSpec((eblk, hc), lambda b: (b, 0)),
                   pl.BlockSpec((1, de), lambda b: (0, 0))],
        out_shape=[jax.ShapeDtypeStruct((e_cnt, hc), f32),
                   jax.ShapeDtypeStruct((1, de), f32)],
        scratch_shapes=[pltpu.VMEM((1, de), f32)],
    )(edge_attr, We[0:1])
    e_layers = [e0] + [
        pl.pallas_call(
            _e_kernel,
            grid=(nbe,),
            in_specs=[pl.BlockSpec((eblk, de), lambda b: (b, 0)),
                      _full((1, de, hc))],
            out_specs=pl.BlockSpec((eblk, hc), lambda b: (b, 0)),
            out_shape=jax.ShapeDtypeStruct((e_cnt, hc), f32),
        )(edge_attr, We[l : l + 1])
        for l in range(1, L)
    ]

    weight_specs = [
        _full((1, de)),
        _full((1, de, hc)),
        _full((d, hc)),
        _full((1, hc)),
        _full((d, hc)),
        _full((1, hc)),
        _full((1, hc)),
        _full((hc, hc)),
        _full((hc, C)),
    ]
    proj_outs = (
        [jax.ShapeDtypeStruct((n, d), f32)] * 3
        + [jax.ShapeDtypeStruct((n, hc), f32),
           jax.ShapeDtypeStruct((n, C), f32)]
    )
    proj_out_specs = [_nodeblk(d)] * 3 + [_nodeblk(hc), _nodeblk(C)]

    h, xl, xr, init_num, init_den = pl.pallas_call(
        functools.partial(_in_proj_kernel, inv_e=inv_e),
        grid=(nblk,),
        in_specs=[_nodeblk(d), _full((d, d)), _full((1, d))] + weight_specs,
        out_specs=proj_out_specs,
        out_shape=proj_outs,
    )(x, W_in, b_in.reshape(1, d), ea_sum, We[0:1], Wl[0], bl[0:1], Wr[0],
      br[0:1], att[0].reshape(1, hc), Pbd, Esel)

    zeros_num = jnp.zeros((n, hc), f32)
    zeros_den = jnp.zeros((n, C), f32)
    for l in range(L):
        acc_num, acc_den = _edge_pass(xl, xr, e_layers[l], src, dst, init_num,
                                      init_den, zeros_num, zeros_den,
                                      att[l], n, H, C)

        out_feat, sums = pl.pallas_call(
            functools.partial(_segnorm_kernel, nblk=nblk),
            grid=(nblk,),
            in_specs=[_pairblk(hc), _pairblk(C), _full((1, hc)),
                      _full((C, hc))],
            out_specs=[_nodeblk(hc), pl.BlockSpec((2, hc), lambda b: (0, 0))],
            out_shape=[jax.ShapeDtypeStruct((n, hc), f32),
                       jax.ShapeDtypeStruct((2, hc), f32)],
            scratch_shapes=[pltpu.VMEM((1, hc), f32)] * 2,
        )(acc_num, acc_den, bias_conv[l : l + 1], Bh)

        if l < L - 1:
            h, xl, xr, init_num, init_den = pl.pallas_call(
                functools.partial(_norm_proj_kernel, inv_n=inv_n, inv_e=inv_e),
                grid=(nblk,),
                in_specs=[_nodeblk(hc), pl.BlockSpec((2, hc), lambda b: (0, 0)),
                          _nodeblk(d), _full((1, d)), _full((1, d)),
                          _full((1, d))] + weight_specs,
                out_specs=proj_out_specs,
                out_shape=proj_outs,
            )(out_feat, sums, h, gn_w[l : l + 1], gn_b[l : l + 1],
              gn_ms[l : l + 1], ea_sum, We[l + 1 : l + 2], Wl[l + 1],
              bl[l + 1 : l + 2], Wr[l + 1], br[l + 1 : l + 2],
              att[l + 1].reshape(1, hc), Pbd, Esel)
        else:
            res = pl.pallas_call(
                functools.partial(_norm_head_kernel, inv_n=inv_n, nblk=nblk),
                grid=(nblk,),
                in_specs=[_nodeblk(hc), pl.BlockSpec((2, hc), lambda b: (0, 0)),
                          _nodeblk(d), _full((1, d)), _full((1, d)),
                          _full((1, d)), _full((d, d // 2)),
                          _full((1, d // 2)), _full((d // 2, 4)),
                          _full((1, 4))],
                out_specs=pl.BlockSpec((1, 4), lambda b: (0, 0)),
                out_shape=jax.ShapeDtypeStruct((1, 4), f32),
                scratch_shapes=[pltpu.VMEM((1, d), f32)],
            )(out_feat, sums, h, gn_w[l : l + 1], gn_b[l : l + 1],
              gn_ms[l : l + 1], Wp1, bp1.reshape(1, -1), Wp2,
              bp2.reshape(1, -1))
    return res


# speedup vs baseline: 2.2124x; 1.3182x over previous
import functools

import jax
import jax.numpy as jnp
import numpy as np
from jax.experimental import pallas as pl
from jax.experimental.pallas import tpu as pltpu
from jax.experimental.pallas import tpu_sc as plsc


def _proj_body(h, easum, We_l, Wl, bl, Wr, br, attf, Pbd, Esel, inv_e,
               xl_ref, xr_ref, accn_ref, accd_ref):
    xl = h @ Wl + bl
    xr = h @ Wr + br
    em = (easum * inv_e) @ We_l
    t = xl + xr + em
    m = jnp.maximum(t, 0.2 * t)
    s = (m * attf) @ Pbd
    w = jnp.exp(s)
    xl_ref[...] = xl
    xr_ref[...] = xr
    accn_ref[...] = w * xl
    accd_ref[...] = w @ Esel


def _in_proj_kernel(x_ref, Win_ref, bin_ref, easum_ref, We_ref, Wl_ref,
                    bl_ref, Wr_ref, br_ref, attf_ref, Pbd_ref, Esel_ref,
                    h_ref, xl_ref, xr_ref, accn_ref, accd_ref, *, inv_e):
    h = x_ref[...] @ Win_ref[...] + bin_ref[...]
    h_ref[...] = h
    _proj_body(h, easum_ref[...], We_ref[0], Wl_ref[...], bl_ref[...],
               Wr_ref[...], br_ref[...], attf_ref[...], Pbd_ref[...],
               Esel_ref[...], inv_e, xl_ref, xr_ref, accn_ref, accd_ref)


def _e_kernel(ea_ref, We_ref, out_ref):
    out_ref[...] = ea_ref[...] @ We_ref[0]


def _e0_kernel(ea_ref, We_ref, out_ref, sum_ref, s1, *, nblk):
    ea = ea_ref[...]
    out_ref[...] = ea @ We_ref[0]

    @pl.when(pl.program_id(0) == 0)
    def _():
        s1[...] = jnp.zeros_like(s1)

    s1[...] += jnp.sum(ea, axis=0, keepdims=True)

    @pl.when(pl.program_id(0) == nblk - 1)
    def _():
        sum_ref[...] = s1[...]


def _segnorm_kernel(num_ref, den_ref, bias_ref, Bh_ref, out_ref, sums_ref,
                    s1, s2, *, nblk):
    num = num_ref[0] + num_ref[1]
    den = (den_ref[0] + den_ref[1]) @ Bh_ref[...]
    out = num / (den + 1e-16) + bias_ref[...]
    out_ref[...] = out

    @pl.when(pl.program_id(0) == 0)
    def _():
        s1[...] = jnp.zeros_like(s1)
        s2[...] = jnp.zeros_like(s2)

    s1[...] += jnp.sum(out, axis=0, keepdims=True)
    s2[...] += jnp.sum(out * out, axis=0, keepdims=True)

    @pl.when(pl.program_id(0) == nblk - 1)
    def _():
        sums_ref[...] = jnp.concatenate([s1[...], s2[...]], axis=0)


def _graphnorm(out, sums, gnw, gnb, gnms, hprev, inv_n):
    mean = sums[0:1] * inv_n
    ex2 = sums[1:2] * inv_n
    var = ex2 - (2.0 * gnms - gnms * gnms) * mean * mean
    outc = out - gnms * mean
    outn = gnw * outc / jnp.sqrt(var + 1e-5) + gnb
    return jnp.maximum(outn, 0.0) + 0.1 * hprev


def _norm_proj_kernel(out_ref, sums_ref, hprev_ref, gnw_ref, gnb_ref,
                      gnms_ref, easum_ref, We_ref, Wl_ref, bl_ref, Wr_ref,
                      br_ref, attf_ref, Pbd_ref, Esel_ref,
                      h_ref, xl_ref, xr_ref, accn_ref, accd_ref,
                      *, inv_n, inv_e):
    h = _graphnorm(out_ref[...], sums_ref[...], gnw_ref[...], gnb_ref[...],
                   gnms_ref[...], hprev_ref[...], inv_n)
    h_ref[...] = h
    _proj_body(h, easum_ref[...], We_ref[0], Wl_ref[...], bl_ref[...],
               Wr_ref[...], br_ref[...], attf_ref[...], Pbd_ref[...],
               Esel_ref[...], inv_e, xl_ref, xr_ref, accn_ref, accd_ref)


def _norm_head_kernel(out_ref, sums_ref, hprev_ref, gnw_ref, gnb_ref,
                      gnms_ref, Wp1_ref, bp1_ref, Wp2_ref, bp2_ref,
                      res_ref, hsum, *, inv_n, nblk):
    h = _graphnorm(out_ref[...], sums_ref[...], gnw_ref[...], gnb_ref[...],
                   gnms_ref[...], hprev_ref[...], inv_n)

    @pl.when(pl.program_id(0) == 0)
    def _():
        hsum[...] = jnp.zeros_like(hsum)

    hsum[...] += jnp.sum(h, axis=0, keepdims=True)

    @pl.when(pl.program_id(0) == nblk - 1)
    def _():
        g = hsum[...] * inv_n
        z = jnp.maximum(g @ Wp1_ref[...] + bp1_ref[...], 0.0)
        res_ref[...] = z @ Wp2_ref[...] + bp2_ref[...]


_CH = 40
_NWORKERS = 32


def _edge_sc_body(n, e_cnt, hc, H, C,
                  src_hbm, dst_hbm, xl_hbm, xr_hbm, e_hbm, att_hbm,
                  initn_hbm, initd_hbm, zn_hbm, zd_hbm,
                  outn_hbm, outd_hbm,
                  srcv, dstv, glv, grv, ev, denv, attv, accn, accd,
                  sem_i, sem_gl, sem_gr, sem_e, sem_s):
    c = jax.lax.axis_index("c")
    s = jax.lax.axis_index("s")
    wid = s * 2 + c
    rpt = n // 16
    r0 = s * rpt

    @pl.when(c == 0)
    def _():
        pltpu.sync_copy(initn_hbm.at[pl.ds(r0, rpt)], accn.at[pl.ds(r0, rpt)])
        pltpu.sync_copy(initd_hbm.at[pl.ds(r0, rpt)], accd.at[pl.ds(r0, rpt)])

    @pl.when(c == 1)
    def _():
        pltpu.sync_copy(zn_hbm.at[pl.ds(r0, rpt)], accn.at[pl.ds(r0, rpt)])
        pltpu.sync_copy(zd_hbm.at[pl.ds(r0, rpt)], accd.at[pl.ds(r0, rpt)])

    pltpu.sync_copy(att_hbm, attv)
    plsc.subcore_barrier()

    per_w = e_cnt // _NWORKERS
    base_w = wid * per_w
    iota16 = jax.lax.iota(jnp.int32, C)

    nchunks = per_w // _CH

    def _idxfetch(kk, qq):
        base = base_w + kk * _CH
        pltpu.async_copy(src_hbm.at[pl.ds(base, _CH)], srcv.at[qq],
                         sem_i.at[qq])
        pltpu.async_copy(dst_hbm.at[pl.ds(base, _CH)], dstv.at[qq],
                         sem_i.at[qq])

    def _wait_idx(qq):
        pltpu.make_async_copy(src_hbm.at[pl.ds(base_w, _CH)], srcv.at[qq],
                              sem_i.at[qq]).wait()
        pltpu.make_async_copy(dst_hbm.at[pl.ds(base_w, _CH)], dstv.at[qq],
                              sem_i.at[qq]).wait()

    def _gfetch(kk, rr, pp, qq):
        base = base_w + kk * _CH
        pltpu.async_copy(xl_hbm.at[srcv.at[qq]], glv.at[rr], sem_gl.at[rr])
        pltpu.async_copy(xr_hbm.at[dstv.at[qq]], grv.at[pp], sem_gr.at[pp])
        pltpu.async_copy(e_hbm.at[pl.ds(base, _CH)], ev.at[pp],
                         sem_e.at[pp])

    def _drain_scatter(rr, qq):
        pltpu.make_async_copy(glv.at[rr], accn.at[dstv.at[qq]],
                              sem_s.at[rr]).wait()
        pltpu.make_async_copy(denv.at[rr], accd.at[dstv.at[qq]],
                              sem_s.at[rr]).wait()

    _idxfetch(0, 0)
    _idxfetch(1, 1)
    _wait_idx(0)
    _gfetch(0, 0, 0, 0)

    @pl.loop(0, nchunks)
    def _chunk(k):
        r = k % 3
        p = k % 2
        q = k % 4
        rn = (k + 1) % 3
        pn = (k + 1) % 2
        qn = (k + 1) % 4

        @pl.when(k + 1 < nchunks)
        def _():
            @pl.when(k >= 2)
            def _():
                _drain_scatter(rn, (k + 2) % 4)
            _wait_idx(qn)
            _gfetch(k + 1, rn, pn, qn)

        @pl.when(k + 2 < nchunks)
        def _():
            _idxfetch(k + 2, (k + 2) % 4)

        pltpu.make_async_copy(xl_hbm.at[srcv.at[q]], glv.at[r],
                              sem_gl.at[r]).wait()
        pltpu.make_async_copy(xr_hbm.at[dstv.at[q]], grv.at[p],
                              sem_gr.at[p]).wait()
        pltpu.make_async_copy(e_hbm.at[pl.ds(base_w, _CH)], ev.at[p],
                              sem_e.at[p]).wait()

        @plsc.parallel_loop(0, _CH, unroll=8)
        def _edge(i):
            gl = [glv[r, i, pl.ds(h * C, C)] for h in range(H)]
            gr = [grv[p, i, pl.ds(h * C, C)] for h in range(H)]
            eh = [ev[p, i, pl.ds(h * C, C)] for h in range(H)]
            wv = []
            for h in range(H):
                t = gl[h] + gr[h] + eh[h]
                m = jnp.maximum(t, 0.2 * t)
                alpha = jnp.sum(m * attv[h])
                wv.append(jnp.exp(jax.lax.broadcast_in_dim(alpha, (C,), ())))
            den_parts = [jnp.where(iota16 == h, wv[h], 0.0) for h in range(H)]
            while len(den_parts) > 1:
                den_parts = [a + b for a, b in
                             zip(den_parts[::2], den_parts[1::2])]
            for h in range(H):
                glv[r, i, pl.ds(h * C, C)] = gl[h] * wv[h]
            denv[r, i, :] = den_parts[0]

        pltpu.async_copy(glv.at[r], accn.at[dstv.at[r]], sem_s.at[r],
                         add=True)
        pltpu.async_copy(denv.at[r], accd.at[dstv.at[r]], sem_s.at[r],
                         add=True)

    for t in (3, 2, 1):
        _drain_scatter((nchunks - t) % 3, (nchunks - t) % 4)
    plsc.subcore_barrier()
    pltpu.sync_copy(accn.at[pl.ds(r0, rpt)], outn_hbm.at[c, pl.ds(r0, rpt)])
    pltpu.sync_copy(accd.at[pl.ds(r0, rpt)], outd_hbm.at[c, pl.ds(r0, rpt)])


def _edge_pass(xl, xr, e_l, src, dst, init_num, init_den,
               zeros_num, zeros_den, att_l, n, H, C):
    hc = H * C
    e_cnt = src.shape[0]
    f32 = jnp.float32
    mesh = plsc.VectorSubcoreMesh(core_axis_name="c", subcore_axis_name="s")

    edge_kernel = functools.partial(
        pl.kernel,
        out_type=[jax.ShapeDtypeStruct((2, n, hc), f32),
                  jax.ShapeDtypeStruct((2, n, C), f32)],
        mesh=mesh,
        scratch_types=[
            pltpu.VMEM((4, _CH), jnp.int32),
            pltpu.VMEM((4, _CH), jnp.int32),
            pltpu.VMEM((3, _CH, hc), f32),
            pltpu.VMEM((2, _CH, hc), f32),
            pltpu.VMEM((2, _CH, hc), f32),
            pltpu.VMEM((3, _CH, C), f32),
            pltpu.VMEM((H, C), f32),
            pltpu.VMEM_SHARED((n, hc), f32),
            pltpu.VMEM_SHARED((n, C), f32),
            pltpu.SemaphoreType.DMA((4,)),
            pltpu.SemaphoreType.DMA((3,)),
            pltpu.SemaphoreType.DMA((2,)),
            pltpu.SemaphoreType.DMA((2,)),
            pltpu.SemaphoreType.DMA((3,)),
        ],
        compiler_params=pltpu.CompilerParams(
            use_tc_tiling_on_sc=False, needs_layout_passes=False),
    )(functools.partial(_edge_sc_body, n, e_cnt, hc, H, C))

    return edge_kernel(src, dst, xl, xr, e_l, att_l, init_num, init_den,
                       zeros_num, zeros_den)


_BN = 400


def _full(shape):
    return pl.BlockSpec(shape, lambda *args: tuple(0 for _ in shape))


def _nodeblk(width):
    return pl.BlockSpec((_BN, width), lambda b: (b, 0))


def _pairblk(width):
    return pl.BlockSpec((2, _BN, width), lambda b: (0, b, 0))


def kernel(x, edge_index, edge_attr, W_in, b_in, Wl, bl, Wr, br, We, att,
           bias_conv, gn_w, gn_b, gn_ms, Wp1, bp1, Wp2, bp2):
    n, d = x.shape
    e_cnt, de = edge_attr.shape
    L, _, hc = Wl.shape
    H, C = att.shape[1], att.shape[2]
    nblk = n // _BN
    inv_n = 1.0 / n
    inv_e = 1.0 / e_cnt

    f32 = jnp.float32
    ii = np.arange(hc)
    Pbd = jnp.asarray((ii[:, None] // C == ii[None, :] // C), dtype=f32)
    Esel = jnp.zeros((hc, C), f32).at[np.arange(H) * C, np.arange(H)].set(1.0)
    Bh = jnp.asarray((np.arange(C)[:, None] == ii[None, :] // C), dtype=f32)

    src = edge_index[0]
    dst = edge_index[1]

    eblk = 8000
    nbe = e_cnt // eblk
    e0, ea_sum = pl.pallas_call(
        functools.partial(_e0_kernel, nblk=nbe),
        grid=(nbe,),
        in_specs=[pl.BlockSpec((eblk, de), lambda b: (b, 0)),
                  _full((1, de, hc))],
        out_specs=[pl.BlockSpec((eblk, hc), lambda b: (b, 0)),
                   pl.BlockSpec((1, de), lambda b: (0, 0))],
        out_shape=[jax.ShapeDtypeStruct((e_cnt, hc), f32),
                   jax.ShapeDtypeStruct((1, de), f32)],
        scratch_shapes=[pltpu.VMEM((1, de), f32)],
    )(edge_attr, We[0:1])
    e_layers = [e0] + [
        pl.pallas_call(
            _e_kernel,
            grid=(nbe,),
            in_specs=[pl.BlockSpec((eblk, de), lambda b: (b, 0)),
                      _full((1, de, hc))],
            out_specs=pl.BlockSpec((eblk, hc), lambda b: (b, 0)),
            out_shape=jax.ShapeDtypeStruct((e_cnt, hc), f32),
        )(edge_attr, We[l : l + 1])
        for l in range(1, L)
    ]

    weight_specs = [
        _full((1, de)),
        _full((1, de, hc)),
        _full((d, hc)),
        _full((1, hc)),
        _full((d, hc)),
        _full((1, hc)),
        _full((1, hc)),
        _full((hc, hc)),
        _full((hc, C)),
    ]
    proj_outs = (
        [jax.ShapeDtypeStruct((n, d), f32)] * 3
        + [jax.ShapeDtypeStruct((n, hc), f32),
           jax.ShapeDtypeStruct((n, C), f32)]
    )
    proj_out_specs = [_nodeblk(d)] * 3 + [_nodeblk(hc), _nodeblk(C)]

    h, xl, xr, init_num, init_den = pl.pallas_call(
        functools.partial(_in_proj_kernel, inv_e=inv_e),
        grid=(nblk,),
        in_specs=[_nodeblk(d), _full((d, d)), _full((1, d))] + weight_specs,
        out_specs=proj_out_specs,
        out_shape=proj_outs,
    )(x, W_in, b_in.reshape(1, d), ea_sum, We[0:1], Wl[0], bl[0:1], Wr[0],
      br[0:1], att[0].reshape(1, hc), Pbd, Esel)

    zeros_num = jnp.zeros((n, hc), f32)
    zeros_den = jnp.zeros((n, C), f32)
    for l in range(L):
        acc_num, acc_den = _edge_pass(xl, xr, e_layers[l], src, dst, init_num,
                                      init_den, zeros_num, zeros_den,
                                      att[l], n, H, C)

        out_feat, sums = pl.pallas_call(
            functools.partial(_segnorm_kernel, nblk=nblk),
            grid=(nblk,),
            in_specs=[_pairblk(hc), _pairblk(C), _full((1, hc)),
                      _full((C, hc))],
            out_specs=[_nodeblk(hc), pl.BlockSpec((2, hc), lambda b: (0, 0))],
            out_shape=[jax.ShapeDtypeStruct((n, hc), f32),
                       jax.ShapeDtypeStruct((2, hc), f32)],
            scratch_shapes=[pltpu.VMEM((1, hc), f32)] * 2,
        )(acc_num, acc_den, bias_conv[l : l + 1], Bh)

        if l < L - 1:
            h, xl, xr, init_num, init_den = pl.pallas_call(
                functools.partial(_norm_proj_kernel, inv_n=inv_n, inv_e=inv_e),
                grid=(nblk,),
                in_specs=[_nodeblk(hc), pl.BlockSpec((2, hc), lambda b: (0, 0)),
                          _nodeblk(d), _full((1, d)), _full((1, d)),
                          _full((1, d))] + weight_specs,
                out_specs=proj_out_specs,
                out_shape=proj_outs,
            )(out_feat, sums, h, gn_w[l : l + 1], gn_b[l : l + 1],
              gn_ms[l : l + 1], ea_sum, We[l + 1 : l + 2], Wl[l + 1],
              bl[l + 1 : l + 2], Wr[l + 1], br[l + 1 : l + 2],
              att[l + 1].reshape(1, hc), Pbd, Esel)
        else:
            res = pl.pallas_call(
                functools.partial(_norm_head_kernel, inv_n=inv_n, nblk=nblk),
                grid=(nblk,),
                in_specs=[_nodeblk(hc), pl.BlockSpec((2, hc), lambda b: (0, 0)),
                          _nodeblk(d), _full((1, d)), _full((1, d)),
                          _full((1, d)), _full((d, d // 2)),
                          _full((1, d // 2)), _full((d // 2, 4)),
                          _full((1, 4))],
                out_specs=pl.BlockSpec((1, 4), lambda b: (0, 0)),
                out_shape=jax.ShapeDtypeStruct((1, 4), f32),
                scratch_shapes=[pltpu.VMEM((1, d), f32)],
            )(out_feat, sums, h, gn_w[l : l + 1], gn_b[l : l + 1],
              gn_ms[l : l + 1], Wp1, bp1.reshape(1, -1), Wp2,
              bp2.reshape(1, -1))
    return res
